# baseline (speedup 1.0000x reference)
import jax
import jax.numpy as jnp
from jax.experimental import pallas as pl

BN = 400


def _half_max(x_ref, wt):
    bn, hdeg, d_in = x_ref.shape
    x = x_ref[...].reshape(bn * hdeg, d_in).astype(jnp.bfloat16)
    h = jnp.dot(x, wt, preferred_element_type=jnp.float32)
    return jnp.max(h.reshape(bn, hdeg, h.shape[1]), axis=1)


def _agg_kernel(x0_ref, x1_ref, wt_ref, b_ref, out_ref):
    wt = wt_ref[...].astype(jnp.bfloat16)
    m0 = _half_max(x0_ref, wt)
    m1 = _half_max(x1_ref, wt)
    out_ref[...] = jnp.maximum(m0, m1) + b_ref[...]


def kernel(neighbour, W, b):
    n, deg, d_in = neighbour.shape
    d_out = W.shape[0]
    wt = W.T
    b2 = b.reshape(1, d_out)
    grid = (n // BN,)
    hdeg = deg // 2
    return pl.pallas_call(
        _agg_kernel,
        grid=grid,
        in_specs=[
            pl.BlockSpec((BN, hdeg, d_in), lambda i: (i, 0, 0)),
            pl.BlockSpec((BN, hdeg, d_in), lambda i: (i, 1, 0)),
            pl.BlockSpec((d_in, d_out), lambda i: (0, 0)),
            pl.BlockSpec((1, d_out), lambda i: (0, 0)),
        ],
        out_specs=pl.BlockSpec((BN, d_out), lambda i: (i, 0)),
        out_shape=jax.ShapeDtypeStruct((n, d_out), jnp.float32),
    )(neighbour, neighbour, wt, b2)


# speedup vs baseline: 1.0385x; 1.0385x over previous
import jax
import jax.numpy as jnp
from jax.experimental import pallas as pl
from jax.experimental.pallas import tpu as pltpu

BN = 400


def _agg_kernel(x_ref, wt_ref, b_ref, out_ref):
    bn, deg, d_in = x_ref.shape
    x = x_ref[...].reshape(bn * deg, d_in).astype(jnp.bfloat16)
    h = jnp.dot(x, wt_ref[...].astype(jnp.bfloat16),
                preferred_element_type=jnp.float32)
    hr = h.reshape(bn, deg, h.shape[1])
    out_ref[...] = jnp.max(hr, axis=1) + b_ref[...]


def kernel(neighbour, W, b):
    n, deg, d_in = neighbour.shape
    d_out = W.shape[0]
    wt = W.T
    b2 = b.reshape(1, d_out)
    grid = (n // BN,)
    return pl.pallas_call(
        _agg_kernel,
        grid=grid,
        in_specs=[
            pl.BlockSpec((BN, deg, d_in), lambda i: (i, 0, 0)),
            pl.BlockSpec((d_in, d_out), lambda i: (0, 0)),
            pl.BlockSpec((1, d_out), lambda i: (0, 0)),
        ],
        out_specs=pl.BlockSpec((BN, d_out), lambda i: (i, 0)),
        out_shape=jax.ShapeDtypeStruct((n, d_out), jnp.float32),
        compiler_params=pltpu.CompilerParams(
            dimension_semantics=("parallel",),
        ),
    )(neighbour, wt, b2)


# speedup vs baseline: 1.0596x; 1.0203x over previous
import jax
import jax.numpy as jnp
from jax.experimental import pallas as pl
from jax.experimental.pallas import tpu as pltpu

S = 5
BN = 200


def _agg_kernel(*refs):
    x_refs = refs[:S]
    wt_ref, b_ref, out_ref = refs[S:]
    wt = wt_ref[...].astype(jnp.bfloat16)
    b = b_ref[...]
    for j, x_ref in enumerate(x_refs):
        _, bn, deg, d_in = x_ref.shape
        x = x_ref[...].reshape(bn * deg, d_in).astype(jnp.bfloat16)
        h = jnp.dot(x, wt, preferred_element_type=jnp.float32)
        m = jnp.max(h.reshape(bn, deg, h.shape[1]), axis=1)
        out_ref[j, :, :] = m + b


def kernel(neighbour, W, b):
    n, deg, d_in = neighbour.shape
    d_out = W.shape[0]
    ns = n // S
    nv = neighbour.reshape(S, ns, deg, d_in)
    wt = W.T
    b2 = b.reshape(1, d_out)
    grid = (ns // BN,)

    def make_spec(j):
        return pl.BlockSpec((1, BN, deg, d_in), lambda i, j=j: (j, i, 0, 0))

    out = pl.pallas_call(
        _agg_kernel,
        grid=grid,
        in_specs=[make_spec(j) for j in range(S)] + [
            pl.BlockSpec((d_in, d_out), lambda i: (0, 0)),
            pl.BlockSpec((1, d_out), lambda i: (0, 0)),
        ],
        out_specs=pl.BlockSpec((S, BN, d_out), lambda i: (0, i, 0)),
        out_shape=jax.ShapeDtypeStruct((S, ns, d_out), jnp.float32),
        compiler_params=pltpu.CompilerParams(
            dimension_semantics=("arbitrary",),
        ),
    )(*([nv] * S), wt, b2)
    return out.reshape(n, d_out)
